# Initial kernel scaffold; baseline (speedup 1.0000x reference)
import jax
import jax.numpy as jnp
from jax.experimental import pallas as pl


def kernel(x):
    raise NotImplementedError("write your pallas kernel here")


# speedup vs baseline: 1.0801x; 1.0801x over previous
import jax
import jax.numpy as jnp
from jax.experimental import pallas as pl


def _copy_body(x_ref, o_ref):
    o_ref[...] = x_ref[...]


def kernel(x):
    B, N, S, D = x.shape
    return pl.pallas_call(
        _copy_body,
        grid=(B, 2),
        in_specs=[pl.BlockSpec((1, 1, S, D), lambda b, j: (b, 2 * j, 0, 0))],
        out_specs=pl.BlockSpec((1, 1, S, D), lambda b, j: (b, j, 0, 0)),
        out_shape=jax.ShapeDtypeStruct((B, 2, S, D), x.dtype),
    )(x)
